# baseline (speedup 1.0000x reference)
import functools

import jax
import jax.numpy as jnp
from jax import lax
from jax.experimental import pallas as pl
from jax.experimental.pallas import tpu as pltpu
from jax.experimental.pallas import tpu_sc as plsc

N_FACTORS = 32
BATCH = 16384
BLOCK_W = 128
ROWS_PER_BLOCK = BLOCK_W // N_FACTORS
NC = 2
NS = 16
NW = NC * NS
B_PER_W = BATCH // NW
CHUNK = 128
N_CHUNKS = B_PER_W // CHUNK
LANES = 16
GROUPS_PER_CHUNK = CHUNK // LANES


def _gmf_body(user_hbm, item_hbm, uemb_hbm, iemb_hbm, hw_hbm, hb_hbm,
              out_hbm, idx_u, idx_i, blk_u, blk_i, rows_u, rows_i,
              w_v, b_v, out_v, sem_u0, sem_i0, sem_u1, sem_i1):
    wid = lax.axis_index("s") * NC + lax.axis_index("c")
    base = wid * B_PER_W
    sems_u = (sem_u0, sem_u1)
    sems_i = (sem_i0, sem_i1)

    pltpu.sync_copy(hw_hbm.at[0], w_v)
    pltpu.sync_copy(hb_hbm, b_v.at[pl.ds(0, 1)])

    for c in range(N_CHUNKS):
        pltpu.sync_copy(user_hbm.at[pl.ds(base + c * CHUNK, CHUNK)], idx_u.at[c])
        pltpu.sync_copy(item_hbm.at[pl.ds(base + c * CHUNK, CHUNK)], idx_i.at[c])

    for c in range(N_CHUNKS):
        for k in range(CHUNK // LANES):
            sl = pl.ds(k * LANES, LANES)
            blk_u[c, sl] = lax.shift_right_logical(idx_u[c, sl], 2)
            blk_i[c, sl] = lax.shift_right_logical(idx_i[c, sl], 2)

    w0 = w_v[pl.ds(0, LANES)]
    w1 = w_v[pl.ds(LANES, LANES)]
    b = b_v[pl.ds(0, LANES)][0]
    lane = jnp.arange(LANES, dtype=jnp.int32)

    def fire(c):
        p = c & 1
        return (
            pltpu.async_copy(uemb_hbm.at[blk_u.at[c]], rows_u.at[p], sems_u[p]),
            pltpu.async_copy(iemb_hbm.at[blk_i.at[c]], rows_i.at[p], sems_i[p]),
        )

    inflight = {0: fire(0)}
    for c in range(N_CHUNKS):
        if c + 1 < N_CHUNKS:
            inflight[c + 1] = fire(c + 1)
        for cp in inflight.pop(c):
            cp.wait()
        p = c & 1

        def group(g, _, c=c, p=p):
            offs_u = (idx_u[c, pl.ds(g * LANES, LANES)] & 3) * N_FACTORS
            offs_i = (idx_i[c, pl.ds(g * LANES, LANES)] & 3) * N_FACTORS
            acc = jnp.zeros((LANES,), jnp.float32)
            for j in range(LANES):
                r = g * LANES + j
                ou = offs_u[j]
                oi = offs_i[j]
                p0 = rows_u[p, r, pl.ds(ou, LANES)]
                p1 = rows_u[p, r, pl.ds(ou + LANES, LANES)]
                q0 = rows_i[p, r, pl.ds(oi, LANES)]
                q1 = rows_i[p, r, pl.ds(oi + LANES, LANES)]
                s = p0 * q0 * w0 + p1 * q1 * w1
                tot = jnp.sum(s)
                acc = jnp.where(lane == j, tot, acc)
            out_v[pl.ds(c * CHUNK + g * LANES, LANES)] = acc + b
            return 0

        lax.fori_loop(0, GROUPS_PER_CHUNK, group, 0)

    pltpu.sync_copy(out_v, out_hbm.at[pl.ds(base, B_PER_W)])


@jax.jit
def _gmf(user, item, user_emb, item_emb, h_w, h_b):
    uemb = user_emb.reshape(-1, BLOCK_W)
    iemb = item_emb.reshape(-1, BLOCK_W)
    mesh = plsc.VectorSubcoreMesh(core_axis_name="c", subcore_axis_name="s")
    call = functools.partial(
        pl.kernel,
        mesh=mesh,
        out_type=jax.ShapeDtypeStruct((BATCH,), jnp.float32),
        scratch_types=[
            pltpu.VMEM((N_CHUNKS, CHUNK), jnp.int32),
            pltpu.VMEM((N_CHUNKS, CHUNK), jnp.int32),
            pltpu.VMEM((N_CHUNKS, CHUNK), jnp.int32),
            pltpu.VMEM((N_CHUNKS, CHUNK), jnp.int32),
            pltpu.VMEM((2, CHUNK, BLOCK_W), jnp.float32),
            pltpu.VMEM((2, CHUNK, BLOCK_W), jnp.float32),
            pltpu.VMEM((N_FACTORS,), jnp.float32),
            pltpu.VMEM((LANES,), jnp.float32),
            pltpu.VMEM((B_PER_W,), jnp.float32),
            pltpu.SemaphoreType.DMA,
            pltpu.SemaphoreType.DMA,
            pltpu.SemaphoreType.DMA,
            pltpu.SemaphoreType.DMA,
        ],
        compiler_params=pltpu.CompilerParams(needs_layout_passes=False),
    )(_gmf_body)
    return call(user, item, uemb, iemb, h_w, h_b)


def kernel(user, item, user_emb, item_emb, h_w, h_b):
    return _gmf(user, item, user_emb, item_emb, h_w, h_b)


# speedup vs baseline: 3.4547x; 3.4547x over previous
import functools

import jax
import jax.numpy as jnp
from jax import lax
from jax.experimental import pallas as pl
from jax.experimental.pallas import tpu as pltpu
from jax.experimental.pallas import tpu_sc as plsc

N_FACTORS = 32
BATCH = 16384
BLOCK_W = 128
NC = 2
NS = 16
NW = NC * NS
B_PER_W = BATCH // NW
GROUP = 16
N_GROUPS = B_PER_W // GROUP
SUB = 4
N_SUB = GROUP // SUB
LANES = 16


def _gmf_body(user_hbm, item_hbm, uT_hbm, iT_hbm, hw_hbm, hb_hbm,
              out_hbm, idx_u, idx_i, blk_u, blk_i, w_v, b_v, out_v,
              sem0, sem1):
    wid = lax.axis_index("s") * NC + lax.axis_index("c")
    base = wid * B_PER_W
    sems = (sem0, sem1)

    pltpu.sync_copy(hw_hbm.at[0], w_v)
    pltpu.sync_copy(hb_hbm, b_v.at[pl.ds(0, 1)])
    for c in range(B_PER_W // BLOCK_W):
        pltpu.sync_copy(user_hbm.at[pl.ds(base + c * BLOCK_W, BLOCK_W)],
                        idx_u.at[pl.ds(c * BLOCK_W, BLOCK_W)])
        pltpu.sync_copy(item_hbm.at[pl.ds(base + c * BLOCK_W, BLOCK_W)],
                        idx_i.at[pl.ds(c * BLOCK_W, BLOCK_W)])

    w0 = w_v[pl.ds(0, LANES)]
    w1 = w_v[pl.ds(LANES, LANES)]
    b = b_v[pl.ds(0, LANES)][0]
    lane = jnp.arange(LANES, dtype=jnp.int32)
    f_lo = jnp.arange(LANES, dtype=jnp.int32)
    f_hi = f_lo + LANES

    def group(g, _):
        u_vec = idx_u[pl.ds(g * GROUP, GROUP)]
        i_vec = idx_i[pl.ds(g * GROUP, GROUP)]
        tcu = lax.shift_left(lax.shift_right_logical(u_vec, 7), 7)
        tci = lax.shift_left(lax.shift_right_logical(i_vec, 7), 7)
        lnu = u_vec & (BLOCK_W - 1)
        lni = i_vec & (BLOCK_W - 1)

        def fire(s):
            p = s & 1
            cps = []
            for l in range(SUB):
                j = s * SUB + l
                cu = pl.multiple_of(tcu[j], BLOCK_W)
                ci = pl.multiple_of(tci[j], BLOCK_W)
                cps.append(pltpu.async_copy(
                    uT_hbm.at[:, pl.ds(cu, BLOCK_W)], blk_u.at[p, l], sems[p]))
                cps.append(pltpu.async_copy(
                    iT_hbm.at[:, pl.ds(ci, BLOCK_W)], blk_i.at[p, l], sems[p]))
            return cps

        def process(s, acc):
            p = s & 1
            for l in range(SUB):
                j = s * SUB + l
                gu = jnp.broadcast_to(lnu[j], (LANES,))
                gi = jnp.broadcast_to(lni[j], (LANES,))
                p0 = plsc.load_gather(blk_u.at[p, l], [f_lo, gu])
                p1 = plsc.load_gather(blk_u.at[p, l], [f_hi, gu])
                q0 = plsc.load_gather(blk_i.at[p, l], [f_lo, gi])
                q1 = plsc.load_gather(blk_i.at[p, l], [f_hi, gi])
                sprod = p0 * q0 * w0 + p1 * q1 * w1
                tot = jnp.sum(sprod)
                acc = jnp.where(lane == j, tot, acc)
            return acc

        acc = jnp.zeros((LANES,), jnp.float32)
        inflight = {0: fire(0)}
        for s in range(N_SUB):
            if s + 1 < N_SUB:
                inflight[s + 1] = fire(s + 1)
            for cp in inflight.pop(s):
                cp.wait()
            acc = process(s, acc)
        out_v[pl.ds(g * GROUP, GROUP)] = acc + b
        return 0

    lax.fori_loop(0, N_GROUPS, group, 0)
    pltpu.sync_copy(out_v, out_hbm.at[pl.ds(base, B_PER_W)])


@jax.jit
def _gmf(user, item, user_emb, item_emb, h_w, h_b):
    mesh = plsc.VectorSubcoreMesh(core_axis_name="c", subcore_axis_name="s")
    call = functools.partial(
        pl.kernel,
        mesh=mesh,
        out_type=jax.ShapeDtypeStruct((BATCH,), jnp.float32),
        scratch_types=[
            pltpu.VMEM((B_PER_W,), jnp.int32),
            pltpu.VMEM((B_PER_W,), jnp.int32),
            pltpu.VMEM((2, SUB, N_FACTORS, BLOCK_W), jnp.float32),
            pltpu.VMEM((2, SUB, N_FACTORS, BLOCK_W), jnp.float32),
            pltpu.VMEM((N_FACTORS,), jnp.float32),
            pltpu.VMEM((LANES,), jnp.float32),
            pltpu.VMEM((B_PER_W,), jnp.float32),
            pltpu.SemaphoreType.DMA,
            pltpu.SemaphoreType.DMA,
        ],
        compiler_params=pltpu.CompilerParams(needs_layout_passes=False),
    )(_gmf_body)
    return call(user, item, user_emb.T, item_emb.T, h_w, h_b)


def kernel(user, item, user_emb, item_emb, h_w, h_b):
    return _gmf(user, item, user_emb, item_emb, h_w, h_b)
